# baseline (speedup 1.0000x reference)
import functools

import jax
import jax.numpy as jnp
from jax import lax
from jax.experimental import pallas as pl
from jax.experimental.pallas import tpu as pltpu
from jax.experimental.pallas import tpu_sc as plsc

_L = 16
_NC = 2
_NS = 16
_NW = _NC * _NS
_CHUNK = 128
_NCATE = 3

_SC_ITEMS = 4096
_TC_BS = 256


def _maskout_sc_body(bpw, split, x_hbm, label_hbm, out_hbm,
                     label_v, idxf_v, idx2_v, buf_v, cate_s, sem_g, sem_s):
    n_chunks = bpw // _CHUNK + _NCATE - 1
    flat = n_chunks * _CHUNK
    cid = lax.axis_index("c")
    sid = lax.axis_index("s")
    wid = sid * _NC + cid
    base = split + wid * bpw
    lane = lax.iota(jnp.int32, _L)

    pltpu.sync_copy(label_hbm.at[pl.ds(base, bpw)], label_v.at[pl.ds(0, bpw)])

    cstar = label_v[pl.ds(0, _L)][0]
    base_splat = jnp.full((_L,), 0, jnp.int32) + base
    for j in range(flat // _L):
        idxf_v[pl.ds(j * _L, _L)] = base_splat
    for t in range(n_chunks):
        cate_s[t] = cstar

    chunks_used = base - base
    for c in range(_NCATE):
        start = chunks_used * _CHUNK
        off = start
        for j in range(bpw // _L):
            lbl = label_v[pl.ds(j * _L, _L)]
            ids = base + j * _L + lane
            m = lbl == c
            mi = m.astype(jnp.int32)
            pos = off + plsc.cumsum(mi) - mi
            plsc.store_scatter(idxf_v, [pos], ids, mask=m)
            off = off + jnp.sum(mi)
        n_c = off - start

        @pl.when(n_c % _CHUNK != 0)
        def _pad(start=start, off=off, n_c=n_c):
            first = idxf_v[pl.ds(start, _L)][0]
            first_splat = jnp.full((_L,), 0, jnp.int32) + first
            t0 = start + (n_c // _L) * _L
            tail = idxf_v[pl.ds(t0, _L)]
            keep = lane < (off - t0)
            idxf_v[pl.ds(t0, _L)] = jnp.where(keep, tail, first_splat)
            end = start + ((n_c + _CHUNK - 1) // _CHUNK) * _CHUNK

            def fill(u, _):
                idxf_v[pl.ds(t0 + _L + u * _L, _L)] = first_splat
                return 0

            lax.fori_loop(0, (end - (t0 + _L)) // _L, fill, 0)

        nch_c = (n_c + _CHUNK - 1) // _CHUNK

        def wcate(t, _, c=c):
            cate_s[t] = c
            return 0

        lax.fori_loop(chunks_used, chunks_used + nch_c, wcate, 0)
        chunks_used = chunks_used + nch_c

    for t in range(n_chunks):
        for j in range(_CHUNK // _L):
            idx2_v[t, pl.ds(j * _L, _L)] = idxf_v[pl.ds(t * _CHUNK + j * _L, _L)]

    gathers = []
    for t in range(n_chunks):
        c_t = cate_s[t]
        gathers.append(
            pltpu.async_copy(
                x_hbm.at[:, c_t].at[idx2_v.at[t]], buf_v.at[t], sem_g.at[t]
            )
        )
    scatters = []
    for t in range(n_chunks):
        gathers[t].wait()
        scatters.append(
            pltpu.async_copy(buf_v.at[t], out_hbm.at[idx2_v.at[t]], sem_s.at[t])
        )
    for t in range(n_chunks):
        scatters[t].wait()


def _sc_part(x, label, split):
    batch, nr_cate, d = x.shape
    bpw = _SC_ITEMS // _NW
    n_chunks = bpw // _CHUNK + _NCATE - 1
    flat = n_chunks * _CHUNK

    mesh = plsc.VectorSubcoreMesh(core_axis_name="c", subcore_axis_name="s")
    run = pl.kernel(
        functools.partial(_maskout_sc_body, bpw, split),
        out_type=jax.ShapeDtypeStruct((batch, d), x.dtype),
        mesh=mesh,
        scratch_types=[
            pltpu.VMEM((bpw + _L,), jnp.int32),
            pltpu.VMEM((flat,), jnp.int32),
            pltpu.VMEM((n_chunks, _CHUNK), jnp.int32),
            pltpu.VMEM((n_chunks, _CHUNK, d), jnp.float32),
            pltpu.SMEM((8,), jnp.int32),
            pltpu.SemaphoreType.DMA((n_chunks,)),
            pltpu.SemaphoreType.DMA((n_chunks,)),
        ],
        compiler_params=pltpu.CompilerParams(needs_layout_passes=False),
    )
    return run(x, label)


def _tc_select_body(label_ref, x_ref, out_ref):
    lbl = label_ref[0, 0, :]
    sel = lbl[:, None]
    x0 = x_ref[:, 0, :]
    x1 = x_ref[:, 1, :]
    x2 = x_ref[:, 2, :]
    out_ref[...] = jnp.where(sel == 1, x1, jnp.where(sel == 2, x2, x0))


def _tc_part(x, label, split):
    batch, nr_cate, d = x.shape
    n_blocks = split // _TC_BS
    label3 = label.reshape(batch // _TC_BS, 1, _TC_BS)
    return pl.pallas_call(
        _tc_select_body,
        grid=(n_blocks,),
        in_specs=[
            pl.BlockSpec((1, 1, _TC_BS), lambda i: (i, 0, 0)),
            pl.BlockSpec((_TC_BS, nr_cate, d), lambda i: (i, 0, 0)),
        ],
        out_specs=pl.BlockSpec((_TC_BS, d), lambda i: (i, 0)),
        out_shape=jax.ShapeDtypeStruct((split, d), x.dtype),
    )(label3, x)


@jax.jit
def kernel(x, label):
    batch, nr_cate, d = x.shape
    split = batch - _SC_ITEMS
    sc_out = _sc_part(x, label, split)
    tc_out = _tc_part(x, label, split)
    return jnp.concatenate([tc_out, sc_out[split:]], axis=0)


# speedup vs baseline: 1.3837x; 1.3837x over previous
import functools

import jax
import jax.numpy as jnp
from jax import lax
from jax.experimental import pallas as pl
from jax.experimental.pallas import tpu as pltpu
from jax.experimental.pallas import tpu_sc as plsc

_L = 16
_NC = 2
_NS = 16
_NW = _NC * _NS
_CH = 64
_UNROLL = 4


def _maskout_body(bpw, d, x_hbm, label_hbm, out_hbm, label_v, rows3_v, out_v, sems):
    cid = lax.axis_index("c")
    sid = lax.axis_index("s")
    wid = sid * _NC + cid
    base = wid * bpw

    pltpu.sync_copy(label_hbm.at[pl.ds(base, bpw)], label_v.at[pl.ds(0, bpw)])

    n_chunks = bpw // _CH
    copies = [None, None]
    copies[0] = pltpu.async_copy(
        x_hbm.at[pl.ds(base, _CH)], rows3_v.at[0], sems.at[0]
    )
    for k in range(n_chunks):
        par = k % 2
        if k + 1 < n_chunks:
            copies[(k + 1) % 2] = pltpu.async_copy(
                x_hbm.at[pl.ds(base + (k + 1) * _CH, _CH)],
                rows3_v.at[(k + 1) % 2],
                sems.at[(k + 1) % 2],
            )
        copies[par].wait()

        def select(g, _, k=k, par=par):
            for u in range(_UNROLL):
                j = g * _UNROLL + u
                lbl = label_v[pl.ds(k * _CH + j, _L)][0]
                for c8 in range(d // _L):
                    out_v[j, pl.ds(c8 * _L, _L)] = rows3_v[
                        par, j, lbl, pl.ds(c8 * _L, _L)
                    ]
            return 0

        lax.fori_loop(0, _CH // _UNROLL, select, 0)
        pltpu.sync_copy(out_v, out_hbm.at[pl.ds(base + k * _CH, _CH)])


@jax.jit
def kernel(x, label):
    batch, nr_cate, d = x.shape
    bpw = batch // _NW

    mesh = plsc.VectorSubcoreMesh(core_axis_name="c", subcore_axis_name="s")
    run = pl.kernel(
        functools.partial(_maskout_body, bpw, d),
        out_type=jax.ShapeDtypeStruct((batch, d), x.dtype),
        mesh=mesh,
        scratch_types=[
            pltpu.VMEM((bpw + _L,), jnp.int32),
            pltpu.VMEM((2, _CH, nr_cate, d), jnp.float32),
            pltpu.VMEM((_CH, d), jnp.float32),
            pltpu.SemaphoreType.DMA((2,)),
        ],
    )
    return run(x, label)


# speedup vs baseline: 1.4538x; 1.0506x over previous
import functools

import jax
import jax.numpy as jnp
from jax import lax
from jax.experimental import pallas as pl
from jax.experimental.pallas import tpu as pltpu
from jax.experimental.pallas import tpu_sc as plsc

_L = 16
_NC = 2
_NS = 16
_NW = _NC * _NS
_CH = 64
_UNROLL = 4


def _maskout_body(bpw, d, x_hbm, label_hbm, out_hbm, label_v, rows3_v, out_v, sems, osems):
    cid = lax.axis_index("c")
    sid = lax.axis_index("s")
    wid = sid * _NC + cid
    base = wid * bpw

    pltpu.sync_copy(label_hbm.at[pl.ds(base, bpw)], label_v.at[pl.ds(0, bpw)])

    n_chunks = bpw // _CH
    copies = [None, None]
    ocopies = [None, None]
    copies[0] = pltpu.async_copy(
        x_hbm.at[pl.ds(base, _CH)], rows3_v.at[0], sems.at[0]
    )
    for k in range(n_chunks):
        par = k % 2
        if k + 1 < n_chunks:
            copies[(k + 1) % 2] = pltpu.async_copy(
                x_hbm.at[pl.ds(base + (k + 1) * _CH, _CH)],
                rows3_v.at[(k + 1) % 2],
                sems.at[(k + 1) % 2],
            )
        copies[par].wait()
        if ocopies[par] is not None:
            ocopies[par].wait()

        def select(j, _, k=k, par=par):
            lbl = label_v[pl.ds(k * _CH + j, _L)][0]
            for c8 in range(d // _L):
                out_v[par, j, pl.ds(c8 * _L, _L)] = rows3_v[
                    par, j, lbl, pl.ds(c8 * _L, _L)
                ]
            return 0

        lax.fori_loop(0, _CH, select, 0)
        ocopies[par] = pltpu.async_copy(
            out_v.at[par], out_hbm.at[pl.ds(base + k * _CH, _CH)], osems.at[par]
        )
    for par in range(2):
        if ocopies[par] is not None:
            ocopies[par].wait()


@jax.jit
def kernel(x, label):
    batch, nr_cate, d = x.shape
    bpw = batch // _NW

    mesh = plsc.VectorSubcoreMesh(core_axis_name="c", subcore_axis_name="s")
    run = pl.kernel(
        functools.partial(_maskout_body, bpw, d),
        out_type=jax.ShapeDtypeStruct((batch, d), x.dtype),
        mesh=mesh,
        scratch_types=[
            pltpu.VMEM((bpw + _L,), jnp.int32),
            pltpu.VMEM((2, _CH, nr_cate, d), jnp.float32),
            pltpu.VMEM((2, _CH, d), jnp.float32),
            pltpu.SemaphoreType.DMA((2,)),
            pltpu.SemaphoreType.DMA((2,)),
        ],
    )
    return run(x, label)


# speedup vs baseline: 1.4833x; 1.0203x over previous
import functools

import jax
import jax.numpy as jnp
from jax import lax
from jax.experimental import pallas as pl
from jax.experimental.pallas import tpu as pltpu
from jax.experimental.pallas import tpu_sc as plsc

_L = 16
_NC = 2
_NS = 16
_NW = _NC * _NS
_CH = 64
_NCATE = 3


def _maskout_body(bpw, d, x_hbm, label_hbm, out_hbm, label_v, rows3_v, out_v, sems, osems):
    cid = lax.axis_index("c")
    sid = lax.axis_index("s")
    wid = sid * _NC + cid
    base = wid * bpw

    pltpu.sync_copy(label_hbm.at[pl.ds(base, bpw)], label_v.at[pl.ds(0, bpw)])

    n_chunks = bpw // _CH
    copies = [[None] * _NCATE, [None] * _NCATE]
    ocopies = [None, None]
    for c in range(_NCATE):
        copies[0][c] = pltpu.async_copy(
            x_hbm.at[pl.ds(base, _CH), pl.ds(c, 1)], rows3_v.at[0, c], sems.at[0]
        )
    for k in range(n_chunks):
        par = k % 2
        if k + 1 < n_chunks:
            for c in range(_NCATE):
                copies[(k + 1) % 2][c] = pltpu.async_copy(
                    x_hbm.at[pl.ds(base + (k + 1) * _CH, _CH), pl.ds(c, 1)],
                    rows3_v.at[(k + 1) % 2, c],
                    sems.at[(k + 1) % 2],
                )
        for c in range(_NCATE):
            copies[par][c].wait()
        if ocopies[par] is not None:
            ocopies[par].wait()

        def select(j, _, k=k, par=par):
            lbl = label_v[pl.ds(k * _CH + j, _L)][0]
            for c8 in range(d // _L):
                out_v[par, j, pl.ds(c8 * _L, _L)] = rows3_v[
                    par, lbl, j, 0, pl.ds(c8 * _L, _L)
                ]
            return 0

        lax.fori_loop(0, _CH, select, 0)
        ocopies[par] = pltpu.async_copy(
            out_v.at[par], out_hbm.at[pl.ds(base + k * _CH, _CH)], osems.at[par]
        )
    for par in range(2):
        if ocopies[par] is not None:
            ocopies[par].wait()


@jax.jit
def kernel(x, label):
    batch, nr_cate, d = x.shape
    bpw = batch // _NW

    mesh = plsc.VectorSubcoreMesh(core_axis_name="c", subcore_axis_name="s")
    run = pl.kernel(
        functools.partial(_maskout_body, bpw, d),
        out_type=jax.ShapeDtypeStruct((batch, d), x.dtype),
        mesh=mesh,
        scratch_types=[
            pltpu.VMEM((bpw + _L,), jnp.int32),
            pltpu.VMEM((2, nr_cate, _CH, 1, d), jnp.float32),
            pltpu.VMEM((2, _CH, d), jnp.float32),
            pltpu.SemaphoreType.DMA((2,)),
            pltpu.SemaphoreType.DMA((2,)),
        ],
    )
    return run(x, label)
